# baseline (speedup 1.0000x reference)
import math

import jax
import jax.numpy as jnp
from jax import lax
from jax.experimental import pallas as pl
from jax.experimental.pallas import tpu as pltpu
from jax.experimental.pallas import tpu_sc as plsc

_NROWS = 100000
_EMB = 128
_QDIM = 10
_NCLS = 2
_BATCH = 1024
_SEQ = 200
_N = _BATCH * _SEQ
_QP = 16
_PACK = _NROWS * _QP // 128
_SCALE = 1.0 / math.sqrt(float(_EMB))

_BM = 2000


def _qk_body(k_ref, v_ref, qw_ref, qb_ref, qk_ref, v16_ref):
    k = k_ref[...]
    qk = lax.dot_general(k, qw_ref[...], (((1,), (1,)), ((), ())),
                         preferred_element_type=jnp.float32)
    qk_ref[...] = (qk + qb_ref[...]) * _SCALE
    v = v_ref[...]
    v16_ref[...] = jnp.concatenate(
        [v, jnp.zeros((_BM, _QP - _QDIM), jnp.float32)], axis=1)


def _make_tables(K_table, V_table, qw16, qb16):
    return pl.pallas_call(
        _qk_body,
        grid=(_NROWS // _BM,),
        in_specs=[
            pl.BlockSpec((_BM, _EMB), lambda i: (i, 0)),
            pl.BlockSpec((_BM, _QDIM), lambda i: (i, 0)),
            pl.BlockSpec((_QP, _EMB), lambda i: (0, 0)),
            pl.BlockSpec((1, _QP), lambda i: (0, 0)),
        ],
        out_specs=[
            pl.BlockSpec((_BM, _QP), lambda i: (i, 0)),
            pl.BlockSpec((_BM, _QP), lambda i: (i, 0)),
        ],
        out_shape=[
            jax.ShapeDtypeStruct((_NROWS, _QP), jnp.float32),
            jax.ShapeDtypeStruct((_NROWS, _QP), jnp.float32),
        ],
    )(K_table, V_table, qw16, qb16)


_NC = 2
_NS = 16
_NW = _NC * _NS
_NPW = _N // _NW

_CK = 320
_KSTEPS = _NPW // _CK


def _gather_k_body(tab, idx_hbm, out_hbm, idx0, idx1, rb0, rb1,
                   gs0, gs1, ws0, ws1):
    wid = lax.axis_index("s") * _NC + lax.axis_index("c")
    base = wid * _NPW
    idxb, rb, gs, ws = (idx0, idx1), (rb0, rb1), (gs0, gs1), (ws0, ws1)
    g = [None, None]
    w = [None, None]
    pltpu.sync_copy(idx_hbm.at[pl.ds(base, _CK)], idx0)
    g[0] = pltpu.async_copy(tab.at[idx0], rb0, gs0)
    for i in range(_KSTEPS):
        b = i & 1
        nb = 1 - b
        g[b].wait()
        if i + 1 < _KSTEPS:
            if i >= 1:
                w[nb].wait()
            pltpu.sync_copy(
                idx_hbm.at[pl.ds(base + (i + 1) * _CK, _CK)], idxb[nb])
            g[nb] = pltpu.async_copy(tab.at[idxb[nb]], rb[nb], gs[nb])
        w[b] = pltpu.async_copy(
            rb[b], out_hbm.at[pl.ds(base + i * _CK, _CK)], ws[b])
    w[(_KSTEPS - 1) & 1].wait()
    w[(_KSTEPS - 2) & 1].wait()


def _gather_k(K_table, read_flat):
    mesh = plsc.VectorSubcoreMesh(core_axis_name="c", subcore_axis_name="s")
    return pl.kernel(
        _gather_k_body,
        out_type=jax.ShapeDtypeStruct((_N, _EMB), jnp.float32),
        mesh=mesh,
        scratch_types=[
            pltpu.VMEM((_CK,), jnp.int32),
            pltpu.VMEM((_CK,), jnp.int32),
            pltpu.VMEM((_CK, _EMB), jnp.float32),
            pltpu.VMEM((_CK, _EMB), jnp.float32),
            pltpu.SemaphoreType.DMA,
            pltpu.SemaphoreType.DMA,
            pltpu.SemaphoreType.DMA,
            pltpu.SemaphoreType.DMA,
        ],
    )(K_table, read_flat)


_BB = 16
_CV = _BB * _SEQ
_VSTEPS = _NPW // _CV


def _gather_vq_body(vtab, qktab, idx_hbm, korder, vout, qkout, vt_out,
                    idx_v, rows, vtbuf, sem):
    del korder
    wid = lax.axis_index("s") * _NC + lax.axis_index("c")
    base = wid * _NPW
    lane = lax.broadcasted_iota(jnp.int32, (16,), 0)
    for i in range(_VSTEPS):
        off = base + i * _CV
        b0 = wid * (_NPW // _SEQ) + i * _BB
        pltpu.sync_copy(idx_hbm.at[pl.ds(off, _CV)], idx_v)
        pltpu.async_copy(vtab.at[idx_v], rows, sem).wait()
        wv = pltpu.async_copy(rows, vout.at[pl.ds(off, _CV)], sem)

        def transpose_l(l, carry):
            ridx = lane * _SEQ + l
            for q in range(_QDIM):
                cidx = jnp.full((16,), q, jnp.int32)
                vtbuf[q, l, :] = plsc.load_gather(rows, [ridx, cidx])
            return carry

        lax.fori_loop(0, _SEQ, transpose_l, 0)
        wt = pltpu.async_copy(
            vtbuf, vt_out.at[:, :, pl.ds(b0, _BB)], sem)
        wv.wait()
        wt.wait()
    for i in range(_VSTEPS):
        off = base + i * _CV
        pltpu.sync_copy(idx_hbm.at[pl.ds(off, _CV)], idx_v)
        pltpu.async_copy(qktab.at[idx_v], rows, sem).wait()
        pltpu.async_copy(rows, qkout.at[pl.ds(off, _CV)], sem).wait()


def _gather_vq(v16, qk16, read_flat, korder):
    mesh = plsc.VectorSubcoreMesh(core_axis_name="c", subcore_axis_name="s")
    return pl.kernel(
        _gather_vq_body,
        out_type=(
            jax.ShapeDtypeStruct((_N, _QP), jnp.float32),
            jax.ShapeDtypeStruct((_N, _QP), jnp.float32),
            jax.ShapeDtypeStruct((_QDIM, _SEQ, _BATCH), jnp.float32),
        ),
        mesh=mesh,
        scratch_types=[
            pltpu.VMEM((_CV,), jnp.int32),
            pltpu.VMEM((_CV, _QP), jnp.float32),
            pltpu.VMEM((_QDIM, _SEQ, _BB), jnp.float32),
            pltpu.SemaphoreType.DMA,
        ],
        compiler_params=pltpu.CompilerParams(use_tc_tiling_on_sc=False,
                                             needs_layout_passes=False),
    )(v16, qk16, read_flat, korder)


_BCH = 128
_NCH = _BATCH // _BCH
_PR = _SEQ // 8
_BRO = _BCH * _PR


def _attn_body(qk_ref, v_ref, ww_ref, wb_ref, out_ref, m_sc, s_sc):
    ph = pl.program_id(0)
    i = pl.program_id(1)
    qk3 = qk_ref[...].reshape(_BCH, _PR, 128)

    @pl.when(ph == 0)
    def _():
        first = (i == 0).astype(jnp.float32)
        m_old = jnp.where(first > 0, jnp.full((_PR, 128), -1e30), m_sc[...])
        s_old = jnp.where(first > 0, jnp.zeros((_PR, 128)), s_sc[...])
        mc = jnp.max(qk3, axis=0)
        m_new = jnp.maximum(m_old, mc)
        ssum = jnp.sum(jnp.exp(qk3 - m_new[None]), axis=0)
        m_sc[...] = m_new
        s_sc[...] = s_old * jnp.exp(m_old - m_new) + ssum

    @pl.when(ph == 1)
    def _():
        m = m_sc[...]
        rs = 1.0 / s_sc[...]
        w3 = jnp.exp(qk3 - m[None]) * rs[None] * \
            v_ref[...].reshape(_BCH, _PR, 128)
        y = jnp.sum(w3, axis=1)
        cc = lax.broadcasted_iota(jnp.int32, (128, _QP), 0) % _QP
        qq = lax.broadcasted_iota(jnp.int32, (128, _QP), 1)
        sel = (cc == qq).astype(jnp.float32)
        x = lax.dot_general(y, sel, (((1,), (0,)), ((), ())),
                            preferred_element_type=jnp.float32)
        out_ref[...] = lax.dot_general(
            x, ww_ref[...], (((1,), (1,)), ((), ())),
            preferred_element_type=jnp.float32) + wb_ref[...]


def _attn(qkp, vp, ww16, wb2):
    return pl.pallas_call(
        _attn_body,
        grid=(2, _NCH),
        in_specs=[
            pl.BlockSpec((_BRO, 128), lambda ph, i: (i, 0)),
            pl.BlockSpec((_BRO, 128), lambda ph, i: (i, 0)),
            pl.BlockSpec((_NCLS, _QP), lambda ph, i: (0, 0)),
            pl.BlockSpec((1, _NCLS), lambda ph, i: (0, 0)),
        ],
        out_specs=pl.BlockSpec((_BCH, _NCLS), lambda ph, i: (i, 0)),
        out_shape=jax.ShapeDtypeStruct((_BATCH, _NCLS), jnp.float32),
        scratch_shapes=[pltpu.VMEM((_PR, 128), jnp.float32),
                        pltpu.VMEM((_PR, 128), jnp.float32)],
    )(qkp, vp, ww16, wb2)


def kernel(read, K_table, V_table, Q_w, Q_b, W_w, W_b):
    read_flat = read.reshape(_N)
    k_lookup_flat = _gather_k(K_table, read_flat)

    qw16 = jnp.zeros((_QP, _EMB), jnp.float32).at[:_QDIM].set(Q_w)
    qb16 = jnp.zeros((1, _QP), jnp.float32).at[0, :_QDIM].set(Q_b)
    ww16 = jnp.zeros((_NCLS, _QP), jnp.float32).at[:, :_QDIM].set(W_w)
    wb2 = W_b.reshape(1, _NCLS)

    qk16, v16 = _make_tables(K_table, V_table, qw16, qb16)
    vg, qkg, vt = _gather_vq(v16, qk16, read_flat, k_lookup_flat)

    out = _attn(qkg.reshape(_N * _QP // 128, 128),
                vg.reshape(_N * _QP // 128, 128), ww16, wb2)
    k_lookup = k_lookup_flat.reshape(_BATCH, _SEQ, _EMB)
    v_lookup = vt.transpose(2, 1, 0)
    return (out, k_lookup, v_lookup)


# speedup vs baseline: 1.3731x; 1.3731x over previous
import math

import jax
import jax.numpy as jnp
from jax import lax
from jax.experimental import pallas as pl
from jax.experimental.pallas import tpu as pltpu
from jax.experimental.pallas import tpu_sc as plsc

_NROWS = 100000
_EMB = 128
_QDIM = 10
_NCLS = 2
_BATCH = 1024
_SEQ = 200
_N = _BATCH * _SEQ
_QP = 16
_PACK = _NROWS * _QP // 128
_SCALE = 1.0 / math.sqrt(float(_EMB))

_BM = 2000


def _qk_body(k_ref, v_ref, qw_ref, qb_ref, qk_ref, v128_ref):
    k = k_ref[...]
    qk = lax.dot_general(k, qw_ref[...], (((1,), (1,)), ((), ())),
                         preferred_element_type=jnp.float32)
    qk = (qk + qb_ref[...]) * _SCALE
    qk_ref[...] = jnp.concatenate(
        [qk, jnp.zeros((_BM, 128 - _QP), jnp.float32)], axis=1)
    v128_ref[...] = jnp.concatenate(
        [v_ref[...], jnp.zeros((_BM, 128 - _QDIM), jnp.float32)], axis=1)


def _make_tables(K_table, V_table, qw16, qb16):
    return pl.pallas_call(
        _qk_body,
        grid=(_NROWS // _BM,),
        in_specs=[
            pl.BlockSpec((_BM, _EMB), lambda i: (i, 0)),
            pl.BlockSpec((_BM, _QDIM), lambda i: (i, 0)),
            pl.BlockSpec((_QP, _EMB), lambda i: (0, 0)),
            pl.BlockSpec((1, _QP), lambda i: (0, 0)),
        ],
        out_specs=[
            pl.BlockSpec((_BM, 128), lambda i: (i, 0)),
            pl.BlockSpec((_BM, 128), lambda i: (i, 0)),
        ],
        out_shape=[
            jax.ShapeDtypeStruct((_NROWS, 128), jnp.float32),
            jax.ShapeDtypeStruct((_NROWS, 128), jnp.float32),
        ],
    )(K_table, V_table, qw16, qb16)


_NC = 2
_NS = 16
_NW = _NC * _NS
_NPW = _N // _NW

_CK = 320
_KSTEPS = _NPW // _CK


def _gather_k_body(tab, idx_hbm, out_hbm, idx0, idx1, rb0, rb1,
                   gs0, gs1, ws0, ws1):
    wid = lax.axis_index("s") * _NC + lax.axis_index("c")
    base = wid * _NPW
    idxb, rb, gs, ws = (idx0, idx1), (rb0, rb1), (gs0, gs1), (ws0, ws1)
    g = [None, None]
    w = [None, None]
    pltpu.sync_copy(idx_hbm.at[pl.ds(base, _CK)], idx0)
    g[0] = pltpu.async_copy(tab.at[idx0], rb0, gs0)
    for i in range(_KSTEPS):
        b = i & 1
        nb = 1 - b
        g[b].wait()
        if i + 1 < _KSTEPS:
            if i >= 1:
                w[nb].wait()
            pltpu.sync_copy(
                idx_hbm.at[pl.ds(base + (i + 1) * _CK, _CK)], idxb[nb])
            g[nb] = pltpu.async_copy(tab.at[idxb[nb]], rb[nb], gs[nb])
        w[b] = pltpu.async_copy(
            rb[b], out_hbm.at[pl.ds(base + i * _CK, _CK)], ws[b])
    w[(_KSTEPS - 1) & 1].wait()
    w[(_KSTEPS - 2) & 1].wait()


def _gather_k(K_table, read_flat):
    mesh = plsc.VectorSubcoreMesh(core_axis_name="c", subcore_axis_name="s")
    return pl.kernel(
        _gather_k_body,
        out_type=jax.ShapeDtypeStruct((_N, _EMB), jnp.float32),
        mesh=mesh,
        scratch_types=[
            pltpu.VMEM((_CK,), jnp.int32),
            pltpu.VMEM((_CK,), jnp.int32),
            pltpu.VMEM((_CK, _EMB), jnp.float32),
            pltpu.VMEM((_CK, _EMB), jnp.float32),
            pltpu.SemaphoreType.DMA,
            pltpu.SemaphoreType.DMA,
            pltpu.SemaphoreType.DMA,
            pltpu.SemaphoreType.DMA,
        ],
    )(K_table, read_flat)


_BB = 16
_CV = _BB * _SEQ
_VSTEPS = _NPW // _CV


def _gather_vq_body(vtab, qktab, idx_hbm, vout, qkout, vt_out,
                    idx_v, rows, vtbuf, sem):
    wid = lax.axis_index("s") * _NC + lax.axis_index("c")
    base = wid * _NPW
    lane = lax.broadcasted_iota(jnp.int32, (16,), 0)
    for i in range(_VSTEPS):
        off = base + i * _CV
        b0 = wid * (_NPW // _SEQ) + i * _BB
        pltpu.sync_copy(idx_hbm.at[pl.ds(off, _CV)], idx_v)
        pltpu.async_copy(vtab.at[idx_v], rows, sem).wait()
        wv = pltpu.async_copy(rows, vout.at[pl.ds(off, _CV)], sem)

        def transpose_l(l, carry):
            ridx = lane * _SEQ + l
            for q in range(_QDIM):
                cidx = jnp.full((16,), q, jnp.int32)
                vtbuf[q, l, :] = plsc.load_gather(rows, [ridx, cidx])
            return carry

        lax.fori_loop(0, _SEQ, transpose_l, 0)
        wt = pltpu.async_copy(
            vtbuf, vt_out.at[:, :, pl.ds(b0, _BB)], sem)
        wv.wait()
        wt.wait()
    for i in range(_VSTEPS):
        off = base + i * _CV
        pltpu.sync_copy(idx_hbm.at[pl.ds(off, _CV)], idx_v)
        pltpu.async_copy(qktab.at[idx_v], rows, sem).wait()
        pltpu.async_copy(rows, qkout.at[pl.ds(off, _CV)], sem).wait()


def _gather_vq(v16, qk16, read8):
    mesh = plsc.VectorSubcoreMesh(core_axis_name="c", subcore_axis_name="s")
    return pl.kernel(
        _gather_vq_body,
        out_type=(
            jax.ShapeDtypeStruct((_N, _QP), jnp.float32),
            jax.ShapeDtypeStruct((_N, _QP), jnp.float32),
            jax.ShapeDtypeStruct((_QDIM, _SEQ, _BATCH), jnp.float32),
        ),
        mesh=mesh,
        scratch_types=[
            pltpu.VMEM((_CV,), jnp.int32),
            pltpu.VMEM((_CV, _QP), jnp.float32),
            pltpu.VMEM((_QDIM, _SEQ, _BB), jnp.float32),
            pltpu.SemaphoreType.DMA,
        ],
        compiler_params=pltpu.CompilerParams(use_tc_tiling_on_sc=False,
                                             needs_layout_passes=False),
    )(v16, qk16, read8)


_LB = 40
_CB = _LB * _QP
_GC = _SEQ // _LB


def _attn_body(qk_ref, v_ref, ww_ref, wb_ref, out_ref, x_acc):
    i = pl.program_id(0)
    qk = qk_ref[...]
    m = jnp.max(qk, axis=0, keepdims=True)
    e = jnp.exp(qk - m)
    ssum = jnp.sum(e, axis=0, keepdims=True)
    w = (e / ssum) * v_ref[...]
    cc = lax.broadcasted_iota(jnp.int32, (_CB, _QP), 0) % _QP
    qq = lax.broadcasted_iota(jnp.int32, (_CB, _QP), 1)
    sel = (cc == qq).astype(jnp.float32)
    part = lax.dot_general(w, sel, (((1,), (0,)), ((), ())),
                           preferred_element_type=jnp.float32)

    @pl.when(i == 0)
    def _():
        x_acc[...] = jnp.zeros_like(x_acc)

    x_acc[...] += part

    @pl.when(i == _GC - 1)
    def _():
        out_ref[...] = lax.dot_general(
            x_acc[...], ww_ref[...], (((1,), (1,)), ((), ())),
            preferred_element_type=jnp.float32) + wb_ref[...]


def _attn(qkg2d, vg2d, ww16, wb2):
    return pl.pallas_call(
        _attn_body,
        grid=(_GC,),
        in_specs=[
            pl.BlockSpec((_BATCH, _CB), lambda i: (0, i)),
            pl.BlockSpec((_BATCH, _CB), lambda i: (0, i)),
            pl.BlockSpec((_NCLS, _QP), lambda i: (0, 0)),
            pl.BlockSpec((1, _NCLS), lambda i: (0, 0)),
        ],
        out_specs=pl.BlockSpec((_BATCH, _NCLS), lambda i: (0, 0)),
        out_shape=jax.ShapeDtypeStruct((_BATCH, _NCLS), jnp.float32),
        scratch_shapes=[pltpu.VMEM((_BATCH, _QP), jnp.float32)],
    )(qkg2d, vg2d, ww16, wb2)


def kernel(read, K_table, V_table, Q_w, Q_b, W_w, W_b):
    read_flat = read.reshape(_N)
    k_lookup_flat = _gather_k(K_table, read_flat)

    qw16 = jnp.zeros((_QP, _EMB), jnp.float32).at[:_QDIM].set(Q_w)
    qb16 = jnp.zeros((1, _QP), jnp.float32).at[0, :_QDIM].set(Q_b)
    ww16 = jnp.zeros((_NCLS, _QP), jnp.float32).at[:, :_QDIM].set(W_w)
    wb2 = W_b.reshape(1, _NCLS)

    qk128, v128 = _make_tables(K_table, V_table, qw16, qb16)
    vg, qkg, vt = _gather_vq(v128.reshape(_NROWS * 8, _QP),
                             qk128.reshape(_NROWS * 8, _QP),
                             read_flat * 8)

    out = _attn(qkg.reshape(_BATCH, _SEQ * _QP),
                vg.reshape(_BATCH, _SEQ * _QP), ww16, wb2)
    k_lookup = k_lookup_flat.reshape(_BATCH, _SEQ, _EMB)
    v_lookup = vt.transpose(2, 1, 0)
    return (out, k_lookup, v_lookup)


# speedup vs baseline: 1.5177x; 1.1053x over previous
import math

import jax
import jax.numpy as jnp
from jax import lax
from jax.experimental import pallas as pl
from jax.experimental.pallas import tpu as pltpu
from jax.experimental.pallas import tpu_sc as plsc

_NROWS = 100000
_EMB = 128
_QDIM = 10
_NCLS = 2
_BATCH = 1024
_SEQ = 200
_N = _BATCH * _SEQ
_QP = 16
_PACK = _NROWS * _QP // 128
_SCALE = 1.0 / math.sqrt(float(_EMB))

_BM = 2048


def _qk_body(k_ref, vt_ref, qw_ref, qb_ref, qk_ref, v128_ref):
    k = k_ref[...]
    qk = lax.dot_general(k, qw_ref[...], (((1,), (1,)), ((), ())),
                         preferred_element_type=jnp.float32)
    qk = (qk + qb_ref[...]) * _SCALE
    qk_ref[...] = jnp.concatenate(
        [qk, jnp.zeros((_BM, 128 - _QP), jnp.float32)], axis=1)
    qq = lax.broadcasted_iota(jnp.int32, (_QDIM, 128), 0)
    jj = lax.broadcasted_iota(jnp.int32, (_QDIM, 128), 1)
    emb = (qq == jj).astype(jnp.float32)
    v128_ref[...] = lax.dot_general(
        vt_ref[...], emb, (((0,), (0,)), ((), ())),
        preferred_element_type=jnp.float32)


def _make_tables(K_table, V_tableT, qw16, qb16):
    return pl.pallas_call(
        _qk_body,
        grid=(pl.cdiv(_NROWS, _BM),),
        in_specs=[
            pl.BlockSpec((_BM, _EMB), lambda i: (i, 0)),
            pl.BlockSpec((_QDIM, _BM), lambda i: (0, i)),
            pl.BlockSpec((_QP, _EMB), lambda i: (0, 0)),
            pl.BlockSpec((1, _QP), lambda i: (0, 0)),
        ],
        out_specs=[
            pl.BlockSpec((_BM, 128), lambda i: (i, 0)),
            pl.BlockSpec((_BM, 128), lambda i: (i, 0)),
        ],
        out_shape=[
            jax.ShapeDtypeStruct((_NROWS, 128), jnp.float32),
            jax.ShapeDtypeStruct((_NROWS, 128), jnp.float32),
        ],
    )(K_table, V_tableT, qw16, qb16)


_NC = 2
_NS = 16
_NW = _NC * _NS
_NPW = _N // _NW

_CK = 320
_KSTEPS = _NPW // _CK


def _gather_k_body(tab, idx_hbm, out_hbm, idx0, idx1, rb0, rb1,
                   gs0, gs1, ws0, ws1):
    wid = lax.axis_index("s") * _NC + lax.axis_index("c")
    base = wid * _NPW
    idxb, rb, gs, ws = (idx0, idx1), (rb0, rb1), (gs0, gs1), (ws0, ws1)
    g = [None, None]
    w = [None, None]
    pltpu.sync_copy(idx_hbm.at[pl.ds(base, _CK)], idx0)
    g[0] = pltpu.async_copy(tab.at[idx0], rb0, gs0)
    for i in range(_KSTEPS):
        b = i & 1
        nb = 1 - b
        g[b].wait()
        if i + 1 < _KSTEPS:
            if i >= 1:
                w[nb].wait()
            pltpu.sync_copy(
                idx_hbm.at[pl.ds(base + (i + 1) * _CK, _CK)], idxb[nb])
            g[nb] = pltpu.async_copy(tab.at[idxb[nb]], rb[nb], gs[nb])
        w[b] = pltpu.async_copy(
            rb[b], out_hbm.at[pl.ds(base + i * _CK, _CK)], ws[b])
    w[(_KSTEPS - 1) & 1].wait()
    w[(_KSTEPS - 2) & 1].wait()


def _gather_k(K_table, read_flat):
    mesh = plsc.VectorSubcoreMesh(core_axis_name="c", subcore_axis_name="s")
    return pl.kernel(
        _gather_k_body,
        out_type=jax.ShapeDtypeStruct((_N, _EMB), jnp.float32),
        mesh=mesh,
        scratch_types=[
            pltpu.VMEM((_CK,), jnp.int32),
            pltpu.VMEM((_CK,), jnp.int32),
            pltpu.VMEM((_CK, _EMB), jnp.float32),
            pltpu.VMEM((_CK, _EMB), jnp.float32),
            pltpu.SemaphoreType.DMA,
            pltpu.SemaphoreType.DMA,
            pltpu.SemaphoreType.DMA,
            pltpu.SemaphoreType.DMA,
        ],
    )(K_table, read_flat)


_BB = 16
_CV = _BB * _SEQ
_VSTEPS = _NPW // _CV


def _gather_vq_body(vtab, qktab, idx_hbm, vout, qkout, vt_out,
                    idx_v, rows, vtbuf, sem):
    wid = lax.axis_index("s") * _NC + lax.axis_index("c")
    base = wid * _NPW
    lane = lax.broadcasted_iota(jnp.int32, (16,), 0)
    for i in range(_VSTEPS):
        off = base + i * _CV
        b0 = wid * (_NPW // _SEQ) + i * _BB
        pltpu.sync_copy(idx_hbm.at[pl.ds(off, _CV)], idx_v)
        pltpu.async_copy(vtab.at[idx_v], rows, sem).wait()
        wv = pltpu.async_copy(rows, vout.at[pl.ds(off, _CV)], sem)

        def transpose_l(l, carry):
            ridx = lane * _SEQ + l
            for q in range(_QDIM):
                cidx = jnp.full((16,), q, jnp.int32)
                vtbuf[q, l, :] = plsc.load_gather(rows, [ridx, cidx])
            return carry

        lax.fori_loop(0, _SEQ, transpose_l, 0)
        wt = pltpu.async_copy(
            vtbuf, vt_out.at[:, :, pl.ds(b0, _BB)], sem)
        wv.wait()
        wt.wait()
    for i in range(_VSTEPS):
        off = base + i * _CV
        pltpu.sync_copy(idx_hbm.at[pl.ds(off, _CV)], idx_v)
        pltpu.async_copy(qktab.at[idx_v], rows, sem).wait()
        pltpu.async_copy(rows, qkout.at[pl.ds(off, _CV)], sem).wait()


def _gather_vq(v16, qk16, read8):
    mesh = plsc.VectorSubcoreMesh(core_axis_name="c", subcore_axis_name="s")
    return pl.kernel(
        _gather_vq_body,
        out_type=(
            jax.ShapeDtypeStruct((_N, _QP), jnp.float32),
            jax.ShapeDtypeStruct((_N, _QP), jnp.float32),
            jax.ShapeDtypeStruct((_QDIM, _SEQ, _BATCH), jnp.float32),
        ),
        mesh=mesh,
        scratch_types=[
            pltpu.VMEM((_CV,), jnp.int32),
            pltpu.VMEM((_CV, _QP), jnp.float32),
            pltpu.VMEM((_QDIM, _SEQ, _BB), jnp.float32),
            pltpu.SemaphoreType.DMA,
        ],
        compiler_params=pltpu.CompilerParams(use_tc_tiling_on_sc=False,
                                             needs_layout_passes=False),
    )(v16, qk16, read8)


_LB = 40
_CB = _LB * _QP
_GC = _SEQ // _LB


def _attn_body(qk_ref, v_ref, ww_ref, wb_ref, out_ref, x_acc):
    i = pl.program_id(0)
    qk = qk_ref[...]
    m = jnp.max(qk, axis=0, keepdims=True)
    e = jnp.exp(qk - m)
    ssum = jnp.sum(e, axis=0, keepdims=True)
    w = (e / ssum) * v_ref[...]
    cc = lax.broadcasted_iota(jnp.int32, (_CB, _QP), 0) % _QP
    qq = lax.broadcasted_iota(jnp.int32, (_CB, _QP), 1)
    sel = (cc == qq).astype(jnp.float32)
    part = lax.dot_general(w, sel, (((1,), (0,)), ((), ())),
                           preferred_element_type=jnp.float32)

    @pl.when(i == 0)
    def _():
        x_acc[...] = jnp.zeros_like(x_acc)

    x_acc[...] += part

    @pl.when(i == _GC - 1)
    def _():
        out_ref[...] = lax.dot_general(
            x_acc[...], ww_ref[...], (((1,), (1,)), ((), ())),
            preferred_element_type=jnp.float32) + wb_ref[...]


def _attn(qkg2d, vg2d, ww16, wb2):
    return pl.pallas_call(
        _attn_body,
        grid=(_GC,),
        in_specs=[
            pl.BlockSpec((_BATCH, _CB), lambda i: (0, i)),
            pl.BlockSpec((_BATCH, _CB), lambda i: (0, i)),
            pl.BlockSpec((_NCLS, _QP), lambda i: (0, 0)),
            pl.BlockSpec((1, _NCLS), lambda i: (0, 0)),
        ],
        out_specs=pl.BlockSpec((_BATCH, _NCLS), lambda i: (0, 0)),
        out_shape=jax.ShapeDtypeStruct((_BATCH, _NCLS), jnp.float32),
        scratch_shapes=[pltpu.VMEM((_BATCH, _QP), jnp.float32)],
    )(qkg2d, vg2d, ww16, wb2)


def kernel(read, K_table, V_table, Q_w, Q_b, W_w, W_b):
    read_flat = read.reshape(_N)
    k_lookup_flat = _gather_k(K_table, read_flat)

    qw16 = jnp.zeros((_QP, _EMB), jnp.float32).at[:_QDIM].set(Q_w)
    qb16 = jnp.zeros((1, _QP), jnp.float32).at[0, :_QDIM].set(Q_b)
    ww16 = jnp.zeros((_NCLS, _QP), jnp.float32).at[:, :_QDIM].set(W_w)
    wb2 = W_b.reshape(1, _NCLS)

    qk128, v128 = _make_tables(K_table, V_table.T, qw16, qb16)
    vg, qkg, vt = _gather_vq(v128.reshape(_NROWS * 8, _QP),
                             qk128.reshape(_NROWS * 8, _QP),
                             read_flat * 8)

    out = _attn(qkg.reshape(_BATCH, _SEQ * _QP),
                vg.reshape(_BATCH, _SEQ * _QP), ww16, wb2)
    k_lookup = k_lookup_flat.reshape(_BATCH, _SEQ, _EMB)
    v_lookup = vt.transpose(2, 1, 0)
    return (out, k_lookup, v_lookup)


# speedup vs baseline: 1.5585x; 1.0269x over previous
import math

import jax
import jax.numpy as jnp
from jax import lax
from jax.experimental import pallas as pl
from jax.experimental.pallas import tpu as pltpu
from jax.experimental.pallas import tpu_sc as plsc

_NROWS = 100000
_EMB = 128
_QDIM = 10
_NCLS = 2
_BATCH = 1024
_SEQ = 200
_N = _BATCH * _SEQ
_QP = 16
_PACK = _NROWS * _QP // 128
_SCALE = 1.0 / math.sqrt(float(_EMB))

_BM = 2048


def _qk_body(k_ref, vt_ref, qw_ref, qb_ref, qkv_ref):
    k = k_ref[...]
    qk = lax.dot_general(k, qw_ref[...], (((1,), (1,)), ((), ())),
                         preferred_element_type=jnp.float32)
    qk = (qk + qb_ref[...]) * _SCALE
    qq = lax.broadcasted_iota(jnp.int32, (_QDIM, 128), 0)
    jj = lax.broadcasted_iota(jnp.int32, (_QDIM, 128), 1)
    emb = (qq + _QP == jj).astype(jnp.float32)
    v128 = lax.dot_general(vt_ref[...], emb, (((0,), (0,)), ((), ())),
                           preferred_element_type=jnp.float32)
    qkv_ref[...] = v128 + jnp.concatenate(
        [qk, jnp.zeros((_BM, 128 - _QP), jnp.float32)], axis=1)


def _make_tables(K_table, V_tableT, qw16, qb16):
    return pl.pallas_call(
        _qk_body,
        grid=(pl.cdiv(_NROWS, _BM),),
        in_specs=[
            pl.BlockSpec((_BM, _EMB), lambda i: (i, 0)),
            pl.BlockSpec((_QDIM, _BM), lambda i: (0, i)),
            pl.BlockSpec((_QP, _EMB), lambda i: (0, 0)),
            pl.BlockSpec((1, _QP), lambda i: (0, 0)),
        ],
        out_specs=pl.BlockSpec((_BM, 128), lambda i: (i, 0)),
        out_shape=jax.ShapeDtypeStruct((_NROWS, 128), jnp.float32),
    )(K_table, V_tableT, qw16, qb16)


_NC = 2
_NS = 16
_NW = _NC * _NS
_NPW = _N // _NW

_CK = 320
_KSTEPS = _NPW // _CK


def _gather_k_body(tab, idx_hbm, out_hbm, idx0, idx1, rb0, rb1,
                   gs0, gs1, ws0, ws1):
    wid = lax.axis_index("s") * _NC + lax.axis_index("c")
    base = wid * _NPW
    idxb, rb, gs, ws = (idx0, idx1), (rb0, rb1), (gs0, gs1), (ws0, ws1)
    g = [None, None]
    w = [None, None]
    pltpu.sync_copy(idx_hbm.at[pl.ds(base, _CK)], idx0)
    g[0] = pltpu.async_copy(tab.at[idx0], rb0, gs0)
    for i in range(_KSTEPS):
        b = i & 1
        nb = 1 - b
        g[b].wait()
        if i + 1 < _KSTEPS:
            if i >= 1:
                w[nb].wait()
            pltpu.sync_copy(
                idx_hbm.at[pl.ds(base + (i + 1) * _CK, _CK)], idxb[nb])
            g[nb] = pltpu.async_copy(tab.at[idxb[nb]], rb[nb], gs[nb])
        w[b] = pltpu.async_copy(
            rb[b], out_hbm.at[pl.ds(base + i * _CK, _CK)], ws[b])
    w[(_KSTEPS - 1) & 1].wait()
    w[(_KSTEPS - 2) & 1].wait()


def _gather_k(K_table, read_flat):
    mesh = plsc.VectorSubcoreMesh(core_axis_name="c", subcore_axis_name="s")
    return pl.kernel(
        _gather_k_body,
        out_type=jax.ShapeDtypeStruct((_N, _EMB), jnp.float32),
        mesh=mesh,
        scratch_types=[
            pltpu.VMEM((_CK,), jnp.int32),
            pltpu.VMEM((_CK,), jnp.int32),
            pltpu.VMEM((_CK, _EMB), jnp.float32),
            pltpu.VMEM((_CK, _EMB), jnp.float32),
            pltpu.SemaphoreType.DMA,
            pltpu.SemaphoreType.DMA,
            pltpu.SemaphoreType.DMA,
            pltpu.SemaphoreType.DMA,
        ],
    )(K_table, read_flat)


_BB = 16
_CV = _BB * _SEQ
_VSTEPS = _NPW // _CV


def _gather_vq_body(tab, idxv_hbm, idxq_hbm, vout, qkout, vt_out,
                    idx_v, rows, vtbuf, sem):
    wid = lax.axis_index("s") * _NC + lax.axis_index("c")
    base = wid * _NPW
    lane = lax.broadcasted_iota(jnp.int32, (16,), 0)
    for i in range(_VSTEPS):
        off = base + i * _CV
        b0 = wid * (_NPW // _SEQ) + i * _BB
        pltpu.sync_copy(idxv_hbm.at[pl.ds(off, _CV)], idx_v)
        pltpu.async_copy(tab.at[idx_v], rows, sem).wait()
        wv = pltpu.async_copy(rows, vout.at[pl.ds(off, _CV)], sem)

        def transpose_l(l, carry):
            ridx = lane * _SEQ + l
            for q in range(_QDIM):
                cidx = jnp.full((16,), q, jnp.int32)
                vtbuf[q, l, :] = plsc.load_gather(rows, [ridx, cidx])
            return carry

        lax.fori_loop(0, _SEQ, transpose_l, 0)
        wt = pltpu.async_copy(
            vtbuf, vt_out.at[:, :, pl.ds(b0, _BB)], sem)
        wv.wait()
        wt.wait()
    for i in range(_VSTEPS):
        off = base + i * _CV
        pltpu.sync_copy(idxq_hbm.at[pl.ds(off, _CV)], idx_v)
        pltpu.async_copy(tab.at[idx_v], rows, sem).wait()
        pltpu.async_copy(rows, qkout.at[pl.ds(off, _CV)], sem).wait()


def _gather_vq(tab, read8v, read8q):
    mesh = plsc.VectorSubcoreMesh(core_axis_name="c", subcore_axis_name="s")
    return pl.kernel(
        _gather_vq_body,
        out_type=(
            jax.ShapeDtypeStruct((_N, _QP), jnp.float32),
            jax.ShapeDtypeStruct((_N, _QP), jnp.float32),
            jax.ShapeDtypeStruct((_QDIM, _SEQ, _BATCH), jnp.float32),
        ),
        mesh=mesh,
        scratch_types=[
            pltpu.VMEM((_CV,), jnp.int32),
            pltpu.VMEM((_CV, _QP), jnp.float32),
            pltpu.VMEM((_QDIM, _SEQ, _BB), jnp.float32),
            pltpu.SemaphoreType.DMA,
        ],
        compiler_params=pltpu.CompilerParams(use_tc_tiling_on_sc=False,
                                             needs_layout_passes=False),
    )(tab, read8v, read8q)


_LB = 40
_CB = _LB * _QP
_GC = _SEQ // _LB


def _attn_body(qk_ref, v_ref, ww_ref, wb_ref, out_ref, x_acc):
    i = pl.program_id(0)
    qk = qk_ref[...]
    m = jnp.max(qk, axis=0, keepdims=True)
    e = jnp.exp(qk - m)
    ssum = jnp.sum(e, axis=0, keepdims=True)
    w = (e / ssum) * v_ref[...]
    cc = lax.broadcasted_iota(jnp.int32, (_CB, _QP), 0) % _QP
    qq = lax.broadcasted_iota(jnp.int32, (_CB, _QP), 1)
    sel = (cc == qq).astype(jnp.float32)
    part = lax.dot_general(w, sel, (((1,), (0,)), ((), ())),
                           preferred_element_type=jnp.float32)

    @pl.when(i == 0)
    def _():
        x_acc[...] = jnp.zeros_like(x_acc)

    x_acc[...] += part

    @pl.when(i == _GC - 1)
    def _():
        out_ref[...] = lax.dot_general(
            x_acc[...], ww_ref[...], (((1,), (1,)), ((), ())),
            preferred_element_type=jnp.float32) + wb_ref[...]


def _attn(qkg2d, vg2d, ww16, wb2):
    return pl.pallas_call(
        _attn_body,
        grid=(_GC,),
        in_specs=[
            pl.BlockSpec((_BATCH, _CB), lambda i: (0, i)),
            pl.BlockSpec((_BATCH, _CB), lambda i: (0, i)),
            pl.BlockSpec((_NCLS, _QP), lambda i: (0, 0)),
            pl.BlockSpec((1, _NCLS), lambda i: (0, 0)),
        ],
        out_specs=pl.BlockSpec((_BATCH, _NCLS), lambda i: (0, 0)),
        out_shape=jax.ShapeDtypeStruct((_BATCH, _NCLS), jnp.float32),
        scratch_shapes=[pltpu.VMEM((_BATCH, _QP), jnp.float32)],
    )(qkg2d, vg2d, ww16, wb2)


def kernel(read, K_table, V_table, Q_w, Q_b, W_w, W_b):
    read_flat = read.reshape(_N)
    k_lookup_flat = _gather_k(K_table, read_flat)

    qw16 = jnp.zeros((_QP, _EMB), jnp.float32).at[:_QDIM].set(Q_w)
    qb16 = jnp.zeros((1, _QP), jnp.float32).at[0, :_QDIM].set(Q_b)
    ww16 = jnp.zeros((_NCLS, _QP), jnp.float32).at[:, :_QDIM].set(W_w)
    wb2 = W_b.reshape(1, _NCLS)

    qkv128 = _make_tables(K_table, V_table.T, qw16, qb16)
    read8 = read_flat * 8
    vg, qkg, vt = _gather_vq(qkv128.reshape(_NROWS * 8, _QP),
                             read8 + 1, read8)

    out = _attn(qkg.reshape(_BATCH, _SEQ * _QP),
                vg.reshape(_BATCH, _SEQ * _QP), ww16, wb2)
    k_lookup = k_lookup_flat.reshape(_BATCH, _SEQ, _EMB)
    v_lookup = vt.transpose(2, 1, 0)
    return (out, k_lookup, v_lookup)


# speedup vs baseline: 1.5954x; 1.0237x over previous
import math

import jax
import jax.numpy as jnp
from jax import lax
from jax.experimental import pallas as pl
from jax.experimental.pallas import tpu as pltpu
from jax.experimental.pallas import tpu_sc as plsc

_NROWS = 100000
_EMB = 128
_QDIM = 10
_NCLS = 2
_BATCH = 1024
_SEQ = 200
_N = _BATCH * _SEQ
_QP = 16
_PACK = _NROWS * _QP // 128
_SCALE = 1.0 / math.sqrt(float(_EMB))

_BM = 2048


def _qk_body(k_ref, vt_ref, qw_ref, qb_ref, qkv_ref):
    k = k_ref[...]
    qk = lax.dot_general(k, qw_ref[...], (((1,), (1,)), ((), ())),
                         preferred_element_type=jnp.float32)
    qk = (qk + qb_ref[...]) * _SCALE
    qq = lax.broadcasted_iota(jnp.int32, (_QDIM, 128), 0)
    jj = lax.broadcasted_iota(jnp.int32, (_QDIM, 128), 1)
    emb = (qq + _QP == jj).astype(jnp.float32)
    v128 = lax.dot_general(vt_ref[...], emb, (((0,), (0,)), ((), ())),
                           preferred_element_type=jnp.float32)
    qkv_ref[...] = v128 + jnp.concatenate(
        [qk, jnp.zeros((_BM, 128 - _QP), jnp.float32)], axis=1)


def _make_tables(K_table, V_tableT, qw16, qb16):
    return pl.pallas_call(
        _qk_body,
        grid=(pl.cdiv(_NROWS, _BM),),
        in_specs=[
            pl.BlockSpec((_BM, _EMB), lambda i: (i, 0)),
            pl.BlockSpec((_QDIM, _BM), lambda i: (0, i)),
            pl.BlockSpec((_QP, _EMB), lambda i: (0, 0)),
            pl.BlockSpec((1, _QP), lambda i: (0, 0)),
        ],
        out_specs=pl.BlockSpec((_BM, 128), lambda i: (i, 0)),
        out_shape=jax.ShapeDtypeStruct((_NROWS, 128), jnp.float32),
    )(K_table, V_tableT, qw16, qb16)


_NC = 2
_NS = 16
_NW = _NC * _NS
_NPW = _N // _NW

_CK = 320
_KSTEPS = _NPW // _CK


def _gather_k_body(tab, idx_hbm, out_hbm, idx0, idx1, rb0, rb1,
                   gs0, gs1, ws0, ws1):
    wid = lax.axis_index("s") * _NC + lax.axis_index("c")
    base = wid * _NPW
    idxb, rb, gs, ws = (idx0, idx1), (rb0, rb1), (gs0, gs1), (ws0, ws1)
    g = [None, None]
    w = [None, None]
    pltpu.sync_copy(idx_hbm.at[pl.ds(base, _CK)], idx0)
    g[0] = pltpu.async_copy(tab.at[idx0], rb0, gs0)
    for i in range(_KSTEPS):
        b = i & 1
        nb = 1 - b
        g[b].wait()
        if i + 1 < _KSTEPS:
            if i >= 1:
                w[nb].wait()
            pltpu.sync_copy(
                idx_hbm.at[pl.ds(base + (i + 1) * _CK, _CK)], idxb[nb])
            g[nb] = pltpu.async_copy(tab.at[idxb[nb]], rb[nb], gs[nb])
        w[b] = pltpu.async_copy(
            rb[b], out_hbm.at[pl.ds(base + i * _CK, _CK)], ws[b])
    w[(_KSTEPS - 1) & 1].wait()
    w[(_KSTEPS - 2) & 1].wait()


def _gather_k(K_table, read_flat):
    mesh = plsc.VectorSubcoreMesh(core_axis_name="c", subcore_axis_name="s")
    return pl.kernel(
        _gather_k_body,
        out_type=jax.ShapeDtypeStruct((_N, _EMB), jnp.float32),
        mesh=mesh,
        scratch_types=[
            pltpu.VMEM((_CK,), jnp.int32),
            pltpu.VMEM((_CK,), jnp.int32),
            pltpu.VMEM((_CK, _EMB), jnp.float32),
            pltpu.VMEM((_CK, _EMB), jnp.float32),
            pltpu.SemaphoreType.DMA,
            pltpu.SemaphoreType.DMA,
            pltpu.SemaphoreType.DMA,
            pltpu.SemaphoreType.DMA,
        ],
    )(K_table, read_flat)


_BB = 16
_CV = _BB * _SEQ
_VSTEPS = _NPW // _CV


def _gather_vq_body(tab, idxv_hbm, idxq_hbm, vout, qkout, vt_out,
                    idx_v, idx_q, rows, qrows, vtb0, vtb1, sem, qsem):
    wid = lax.axis_index("s") * _NC + lax.axis_index("c")
    base = wid * _NPW
    lane = lax.broadcasted_iota(jnp.int32, (16,), 0)
    half = _SEQ // 4
    for i in range(_VSTEPS):
        off = base + i * _CV
        b0 = wid * (_NPW // _SEQ) + i * _BB
        pltpu.sync_copy(idxv_hbm.at[pl.ds(off, _CV)], idx_v)
        pltpu.sync_copy(idxq_hbm.at[pl.ds(off, _CV)], idx_q)
        gv = pltpu.async_copy(tab.at[idx_v], rows, sem)
        gq = pltpu.async_copy(tab.at[idx_q], qrows, qsem)
        gv.wait()
        wv = pltpu.async_copy(rows, vout.at[pl.ds(off, _CV)], sem)

        def make_part(vtb, l_base):
            def transpose_l(l, carry):
                ridx = lane * _SEQ + (l_base + l)
                for q in range(_QDIM):
                    cidx = jnp.full((16,), q, jnp.int32)
                    vtb[q, l, :] = plsc.load_gather(rows, [ridx, cidx])
                return carry
            lax.fori_loop(0, half, transpose_l, 0)

        vtbs = (vtb0, vtb1)
        wts = [None, None]
        for p in range(4):
            if wts[p & 1] is not None:
                wts[p & 1].wait()
            make_part(vtbs[p & 1], p * half)
            wts[p & 1] = pltpu.async_copy(
                vtbs[p & 1],
                vt_out.at[:, pl.ds(p * half, half), pl.ds(b0, _BB)], sem)
        gq.wait()
        wq = pltpu.async_copy(qrows, qkout.at[pl.ds(off, _CV)], qsem)
        wv.wait()
        wts[0].wait()
        wts[1].wait()
        wq.wait()


def _gather_vq(tab, read8v, read8q):
    mesh = plsc.VectorSubcoreMesh(core_axis_name="c", subcore_axis_name="s")
    return pl.kernel(
        _gather_vq_body,
        out_type=(
            jax.ShapeDtypeStruct((_N, _QP), jnp.float32),
            jax.ShapeDtypeStruct((_N, _QP), jnp.float32),
            jax.ShapeDtypeStruct((_QDIM, _SEQ, _BATCH), jnp.float32),
        ),
        mesh=mesh,
        scratch_types=[
            pltpu.VMEM((_CV,), jnp.int32),
            pltpu.VMEM((_CV,), jnp.int32),
            pltpu.VMEM((_CV, _QP), jnp.float32),
            pltpu.VMEM((_CV, _QP), jnp.float32),
            pltpu.VMEM((_QDIM, _SEQ // 4, _BB), jnp.float32),
            pltpu.VMEM((_QDIM, _SEQ // 4, _BB), jnp.float32),
            pltpu.SemaphoreType.DMA,
            pltpu.SemaphoreType.DMA,
        ],
        compiler_params=pltpu.CompilerParams(use_tc_tiling_on_sc=False,
                                             needs_layout_passes=False),
    )(tab, read8v, read8q)


_LB = 40
_CB = _LB * _QP
_GC = _SEQ // _LB


def _attn_body(qk_ref, v_ref, ww_ref, wb_ref, out_ref, x_acc):
    i = pl.program_id(0)
    qk = qk_ref[...]
    m = jnp.max(qk, axis=0, keepdims=True)
    e = jnp.exp(qk - m)
    ssum = jnp.sum(e, axis=0, keepdims=True)
    w = (e / ssum) * v_ref[...]
    cc = lax.broadcasted_iota(jnp.int32, (_CB, _QP), 0) % _QP
    qq = lax.broadcasted_iota(jnp.int32, (_CB, _QP), 1)
    sel = (cc == qq).astype(jnp.float32)
    part = lax.dot_general(w, sel, (((1,), (0,)), ((), ())),
                           preferred_element_type=jnp.float32)

    @pl.when(i == 0)
    def _():
        x_acc[...] = jnp.zeros_like(x_acc)

    x_acc[...] += part

    @pl.when(i == _GC - 1)
    def _():
        out_ref[...] = lax.dot_general(
            x_acc[...], ww_ref[...], (((1,), (1,)), ((), ())),
            preferred_element_type=jnp.float32) + wb_ref[...]


def _attn(qkg2d, vg2d, ww16, wb2):
    return pl.pallas_call(
        _attn_body,
        grid=(_GC,),
        in_specs=[
            pl.BlockSpec((_BATCH, _CB), lambda i: (0, i)),
            pl.BlockSpec((_BATCH, _CB), lambda i: (0, i)),
            pl.BlockSpec((_NCLS, _QP), lambda i: (0, 0)),
            pl.BlockSpec((1, _NCLS), lambda i: (0, 0)),
        ],
        out_specs=pl.BlockSpec((_BATCH, _NCLS), lambda i: (0, 0)),
        out_shape=jax.ShapeDtypeStruct((_BATCH, _NCLS), jnp.float32),
        scratch_shapes=[pltpu.VMEM((_BATCH, _QP), jnp.float32)],
    )(qkg2d, vg2d, ww16, wb2)


def kernel(read, K_table, V_table, Q_w, Q_b, W_w, W_b):
    read_flat = read.reshape(_N)
    k_lookup_flat = _gather_k(K_table, read_flat)

    qw16 = jnp.zeros((_QP, _EMB), jnp.float32).at[:_QDIM].set(Q_w)
    qb16 = jnp.zeros((1, _QP), jnp.float32).at[0, :_QDIM].set(Q_b)
    ww16 = jnp.zeros((_NCLS, _QP), jnp.float32).at[:, :_QDIM].set(W_w)
    wb2 = W_b.reshape(1, _NCLS)

    qkv128 = _make_tables(K_table, V_table.T, qw16, qb16)
    read8 = read_flat * 8
    vg, qkg, vt = _gather_vq(qkv128.reshape(_NROWS * 8, _QP),
                             read8 + 1, read8)

    out = _attn(qkg.reshape(_BATCH, _SEQ * _QP),
                vg.reshape(_BATCH, _SEQ * _QP), ww16, wb2)
    k_lookup = k_lookup_flat.reshape(_BATCH, _SEQ, _EMB)
    v_lookup = vt.transpose(2, 1, 0)
    return (out, k_lookup, v_lookup)


# speedup vs baseline: 1.6037x; 1.0052x over previous
import math

import jax
import jax.numpy as jnp
from jax import lax
from jax.experimental import pallas as pl
from jax.experimental.pallas import tpu as pltpu
from jax.experimental.pallas import tpu_sc as plsc

_NROWS = 100000
_EMB = 128
_QDIM = 10
_NCLS = 2
_BATCH = 1024
_SEQ = 200
_N = _BATCH * _SEQ
_QP = 16
_PACK = _NROWS * _QP // 128
_SCALE = 1.0 / math.sqrt(float(_EMB))

_BM = 2048


def _qk_body(k_ref, vt_ref, qw_ref, qb_ref, qkv_ref):
    k = k_ref[...]
    qk = lax.dot_general(k, qw_ref[...], (((1,), (1,)), ((), ())),
                         preferred_element_type=jnp.float32)
    qk = (qk + qb_ref[...]) * _SCALE
    qq = lax.broadcasted_iota(jnp.int32, (_QDIM, 128), 0)
    jj = lax.broadcasted_iota(jnp.int32, (_QDIM, 128), 1)
    emb = (qq + _QP == jj).astype(jnp.float32)
    v128 = lax.dot_general(vt_ref[...], emb, (((0,), (0,)), ((), ())),
                           preferred_element_type=jnp.float32)
    qkv_ref[...] = v128 + jnp.concatenate(
        [qk, jnp.zeros((_BM, 128 - _QP), jnp.float32)], axis=1)


def _make_tables(K_table, V_tableT, qw16, qb16):
    return pl.pallas_call(
        _qk_body,
        grid=(pl.cdiv(_NROWS, _BM),),
        in_specs=[
            pl.BlockSpec((_BM, _EMB), lambda i: (i, 0)),
            pl.BlockSpec((_QDIM, _BM), lambda i: (0, i)),
            pl.BlockSpec((_QP, _EMB), lambda i: (0, 0)),
            pl.BlockSpec((1, _QP), lambda i: (0, 0)),
        ],
        out_specs=pl.BlockSpec((_BM, 128), lambda i: (i, 0)),
        out_shape=jax.ShapeDtypeStruct((_NROWS, 128), jnp.float32),
    )(K_table, V_tableT, qw16, qb16)


_NC = 2
_NS = 16
_NW = _NC * _NS
_NPW = _N // _NW

_CK = 400
_KSTEPS = _NPW // _CK


def _gather_k_body(tab, idx_hbm, out_hbm, idx0, idx1, rb0, rb1,
                   gs0, gs1, ws0, ws1):
    wid = lax.axis_index("s") * _NC + lax.axis_index("c")
    base = wid * _NPW
    idxb, rb, gs, ws = (idx0, idx1), (rb0, rb1), (gs0, gs1), (ws0, ws1)
    g = [None, None]
    w = [None, None]
    pltpu.sync_copy(idx_hbm.at[pl.ds(base, _CK)], idx0)
    g[0] = pltpu.async_copy(tab.at[idx0], rb0, gs0)
    for i in range(_KSTEPS):
        b = i & 1
        nb = 1 - b
        g[b].wait()
        if i + 1 < _KSTEPS:
            if i >= 1:
                w[nb].wait()
            pltpu.sync_copy(
                idx_hbm.at[pl.ds(base + (i + 1) * _CK, _CK)], idxb[nb])
            g[nb] = pltpu.async_copy(tab.at[idxb[nb]], rb[nb], gs[nb])
        w[b] = pltpu.async_copy(
            rb[b], out_hbm.at[pl.ds(base + i * _CK, _CK)], ws[b])
    w[(_KSTEPS - 1) & 1].wait()
    w[(_KSTEPS - 2) & 1].wait()


def _gather_k(K_table, read_flat):
    mesh = plsc.VectorSubcoreMesh(core_axis_name="c", subcore_axis_name="s")
    return pl.kernel(
        _gather_k_body,
        out_type=jax.ShapeDtypeStruct((_N, _EMB), jnp.float32),
        mesh=mesh,
        scratch_types=[
            pltpu.VMEM((_CK,), jnp.int32),
            pltpu.VMEM((_CK,), jnp.int32),
            pltpu.VMEM((_CK, _EMB), jnp.float32),
            pltpu.VMEM((_CK, _EMB), jnp.float32),
            pltpu.SemaphoreType.DMA,
            pltpu.SemaphoreType.DMA,
            pltpu.SemaphoreType.DMA,
            pltpu.SemaphoreType.DMA,
        ],
    )(K_table, read_flat)


_BB = 16
_CV = _BB * _SEQ
_VSTEPS = _NPW // _CV


def _gather_vq_body(tab, idxv_hbm, idxq_hbm, vout, qkout, vt_out,
                    idx_v, idx_q, rows, qrows, vtb0, vtb1, sem, qsem):
    wid = lax.axis_index("s") * _NC + lax.axis_index("c")
    base = wid * _NPW
    lane = lax.broadcasted_iota(jnp.int32, (16,), 0)
    half = _SEQ // 4
    for i in range(_VSTEPS):
        off = base + i * _CV
        b0 = wid * (_NPW // _SEQ) + i * _BB
        pltpu.sync_copy(idxv_hbm.at[pl.ds(off, _CV)], idx_v)
        pltpu.sync_copy(idxq_hbm.at[pl.ds(off, _CV)], idx_q)
        gv = pltpu.async_copy(tab.at[idx_v], rows, sem)
        gq = pltpu.async_copy(tab.at[idx_q], qrows, qsem)
        gv.wait()
        wv = pltpu.async_copy(rows, vout.at[pl.ds(off, _CV)], sem)

        def make_part(vtb, l_base):
            def transpose_l(l, carry):
                ridx = lane * _SEQ + (l_base + l)
                for q in range(_QDIM):
                    cidx = jnp.full((16,), q, jnp.int32)
                    vtb[q, l, :] = plsc.load_gather(rows, [ridx, cidx])
                return carry
            lax.fori_loop(0, half, transpose_l, 0)

        vtbs = (vtb0, vtb1)
        wts = [None, None]
        for p in range(4):
            if wts[p & 1] is not None:
                wts[p & 1].wait()
            make_part(vtbs[p & 1], p * half)
            wts[p & 1] = pltpu.async_copy(
                vtbs[p & 1],
                vt_out.at[:, pl.ds(p * half, half), pl.ds(b0, _BB)], sem)
        gq.wait()
        wq = pltpu.async_copy(qrows, qkout.at[pl.ds(off, _CV)], qsem)
        wv.wait()
        wts[0].wait()
        wts[1].wait()
        wq.wait()


def _gather_vq(tab, read8v, read8q):
    mesh = plsc.VectorSubcoreMesh(core_axis_name="c", subcore_axis_name="s")
    return pl.kernel(
        _gather_vq_body,
        out_type=(
            jax.ShapeDtypeStruct((_N, _QP), jnp.float32),
            jax.ShapeDtypeStruct((_N, _QP), jnp.float32),
            jax.ShapeDtypeStruct((_QDIM, _SEQ, _BATCH), jnp.float32),
        ),
        mesh=mesh,
        scratch_types=[
            pltpu.VMEM((_CV,), jnp.int32),
            pltpu.VMEM((_CV,), jnp.int32),
            pltpu.VMEM((_CV, _QP), jnp.float32),
            pltpu.VMEM((_CV, _QP), jnp.float32),
            pltpu.VMEM((_QDIM, _SEQ // 4, _BB), jnp.float32),
            pltpu.VMEM((_QDIM, _SEQ // 4, _BB), jnp.float32),
            pltpu.SemaphoreType.DMA,
            pltpu.SemaphoreType.DMA,
        ],
        compiler_params=pltpu.CompilerParams(use_tc_tiling_on_sc=False,
                                             needs_layout_passes=False),
    )(tab, read8v, read8q)


_LB = 40
_CB = _LB * _QP
_GC = _SEQ // _LB


def _attn_body(qk_ref, v_ref, ww_ref, wb_ref, out_ref, x_acc):
    i = pl.program_id(0)
    qk = qk_ref[...]
    m = jnp.max(qk, axis=0, keepdims=True)
    e = jnp.exp(qk - m)
    ssum = jnp.sum(e, axis=0, keepdims=True)
    w = (e / ssum) * v_ref[...]
    cc = lax.broadcasted_iota(jnp.int32, (_CB, _QP), 0) % _QP
    qq = lax.broadcasted_iota(jnp.int32, (_CB, _QP), 1)
    sel = (cc == qq).astype(jnp.float32)
    part = lax.dot_general(w, sel, (((1,), (0,)), ((), ())),
                           preferred_element_type=jnp.float32)

    @pl.when(i == 0)
    def _():
        x_acc[...] = jnp.zeros_like(x_acc)

    x_acc[...] += part

    @pl.when(i == _GC - 1)
    def _():
        out_ref[...] = lax.dot_general(
            x_acc[...], ww_ref[...], (((1,), (1,)), ((), ())),
            preferred_element_type=jnp.float32) + wb_ref[...]


def _attn(qkg2d, vg2d, ww16, wb2):
    return pl.pallas_call(
        _attn_body,
        grid=(_GC,),
        in_specs=[
            pl.BlockSpec((_BATCH, _CB), lambda i: (0, i)),
            pl.BlockSpec((_BATCH, _CB), lambda i: (0, i)),
            pl.BlockSpec((_NCLS, _QP), lambda i: (0, 0)),
            pl.BlockSpec((1, _NCLS), lambda i: (0, 0)),
        ],
        out_specs=pl.BlockSpec((_BATCH, _NCLS), lambda i: (0, 0)),
        out_shape=jax.ShapeDtypeStruct((_BATCH, _NCLS), jnp.float32),
        scratch_shapes=[pltpu.VMEM((_BATCH, _QP), jnp.float32)],
    )(qkg2d, vg2d, ww16, wb2)


def kernel(read, K_table, V_table, Q_w, Q_b, W_w, W_b):
    read_flat = read.reshape(_N)
    k_lookup_flat = _gather_k(K_table, read_flat)

    qw16 = jnp.zeros((_QP, _EMB), jnp.float32).at[:_QDIM].set(Q_w)
    qb16 = jnp.zeros((1, _QP), jnp.float32).at[0, :_QDIM].set(Q_b)
    ww16 = jnp.zeros((_NCLS, _QP), jnp.float32).at[:, :_QDIM].set(W_w)
    wb2 = W_b.reshape(1, _NCLS)

    qkv128 = _make_tables(K_table, V_table.T, qw16, qb16)
    read8 = read_flat * 8
    vg, qkg, vt = _gather_vq(qkv128.reshape(_NROWS * 8, _QP),
                             read8 + 1, read8)

    out = _attn(qkg.reshape(_BATCH, _SEQ * _QP),
                vg.reshape(_BATCH, _SEQ * _QP), ww16, wb2)
    k_lookup = k_lookup_flat.reshape(_BATCH, _SEQ, _EMB)
    v_lookup = vt.transpose(2, 1, 0)
    return (out, k_lookup, v_lookup)
